# Initial kernel scaffold; baseline (speedup 1.0000x reference)
import jax
import jax.numpy as jnp
from jax.experimental import pallas as pl


def kernel(x, edge_index, W1, b1, W2, b2):
    raise NotImplementedError("write your pallas kernel here")


# speedup vs baseline: 6.5235x; 6.5235x over previous
import functools

import jax
import jax.numpy as jnp
from jax import lax
from jax.experimental import pallas as pl
from jax.experimental.pallas import tpu as pltpu
from jax.experimental.pallas import tpu_sc as plsc

NS = 16
LANES = 16
EBH = 80
EBP = 32


def _sc_mesh():
    return plsc.VectorSubcoreMesh(core_axis_name="c", subcore_axis_name="s",
                                  num_cores=1)


def _make_degree_hist(n, e):
    ept = e // NS
    nch = ept // EBH

    @functools.partial(
        pl.kernel,
        out_type=jax.ShapeDtypeStruct((n,), jnp.float32),
        mesh=_sc_mesh(),
        scratch_types=[
            pltpu.VMEM((EBH,), jnp.int32),
            pltpu.VMEM((EBH,), jnp.float32),
            pltpu.VMEM((n,), jnp.float32),
            pltpu.VMEM_SHARED((n,), jnp.float32),
        ],
    )
    def hist(dst_hbm, out_hbm, idxb, onesb, zb, counts_sh):
        s = lax.axis_index("s")
        for j in range(EBH // LANES):
            onesb[pl.ds(j * LANES, LANES)] = jnp.ones((LANES,), jnp.float32)

        @pl.when(s == 0)
        def _zero():
            def zf(i, carry):
                zb[pl.ds(i * LANES, LANES)] = jnp.zeros((LANES,), jnp.float32)
                return carry
            lax.fori_loop(0, n // LANES, zf, 0)
            pltpu.sync_copy(zb, counts_sh)

        plsc.subcore_barrier()
        base = s * ept

        def body(k, carry):
            pltpu.sync_copy(dst_hbm.at[pl.ds(base + k * EBH, EBH)], idxb)
            pltpu.sync_copy(onesb, counts_sh.at[idxb], add=True)
            return carry
        lax.fori_loop(0, nch, body, 0)

        plsc.subcore_barrier()

        @pl.when(s == 0)
        def _out():
            pltpu.sync_copy(counts_sh, out_hbm)

    return hist


def _make_propagate(n, d, e):
    ept = e // NS
    nch = ept // EBP
    assert nch % 2 == 1, "pipeline below assumes an odd chunk count"
    nh = n // 2
    rows_a = ((nh + NS - 1) // NS + 7) // 8 * 8
    rows_last = nh - rows_a * (NS - 1)

    @functools.partial(
        pl.kernel,
        out_type=jax.ShapeDtypeStruct((n, d), jnp.float32),
        mesh=_sc_mesh(),
        scratch_types=[
            pltpu.VMEM((ept,), jnp.int32),
            pltpu.VMEM((ept,), jnp.int32),
            pltpu.VMEM((EBP,), jnp.int32),
            pltpu.VMEM((EBP,), jnp.int32),
            pltpu.VMEM((EBP,), jnp.int32),
            pltpu.VMEM((EBP,), jnp.int32),
            pltpu.VMEM((EBP, d), jnp.float32),
            pltpu.VMEM((EBP, d), jnp.float32),
            pltpu.VMEM((rows_a, d), jnp.float32),
            pltpu.VMEM_SHARED((nh + 1, d), jnp.float32),
            pltpu.SemaphoreType.DMA,
            pltpu.SemaphoreType.DMA,
        ],
    )
    def prop(hs_hbm, src_hbm, dst_hbm, out_hbm, srcall, dstall,
             sb0, sb1, db0, db1, rb0, rb1, zb, agg_sh, sem0, sem1):
        s = lax.axis_index("s")
        base = s * ept
        pltpu.sync_copy(src_hbm.at[pl.ds(base, ept)], srcall)
        pltpu.sync_copy(dst_hbm.at[pl.ds(base, ept)], dstall)

        def zf(i, carry):
            for j in range(d // LANES):
                zb[i, pl.ds(j * LANES, LANES)] = jnp.zeros((LANES,), jnp.float32)
            return carry
        lax.fori_loop(0, rows_a, zf, 0)

        def prep_src(k, sb):
            for j in range(EBP // LANES):
                sb[pl.ds(j * LANES, LANES)] = srcall[pl.ds(k * EBP + j * LANES,
                                                           LANES)]

        def prep_dst(k, db, lo):
            for j in range(EBP // LANES):
                v = dstall[pl.ds(k * EBP + j * LANES, LANES)]
                local = v - lo
                ok = (local >= 0) & (local < nh)
                db[pl.ds(j * LANES, LANES)] = jnp.where(ok, local, nh)

        for p in range(2):
            lo = p * nh

            @pl.when(s < NS - 1)
            def _zmain():
                pltpu.sync_copy(zb, agg_sh.at[pl.ds(s * rows_a, rows_a)])

            @pl.when(s == NS - 1)
            def _zlast():
                pltpu.sync_copy(zb.at[pl.ds(0, rows_last)],
                                agg_sh.at[pl.ds((NS - 1) * rows_a, rows_last)])

            plsc.subcore_barrier()

            prep_src(0, sb0)
            pltpu.async_copy(hs_hbm.at[sb0], rb0, sem0)

            def body(k2, carry):
                k0 = 2 * k2
                k1 = k0 + 1
                prep_src(k1, sb1)
                pltpu.async_copy(hs_hbm.at[sb1], rb1, sem1)
                pltpu.make_async_copy(hs_hbm.at[sb0], rb0, sem0).wait()
                prep_dst(k0, db0, lo)
                pltpu.sync_copy(rb0, agg_sh.at[db0], add=True)
                prep_src(k0 + 2, sb0)
                pltpu.async_copy(hs_hbm.at[sb0], rb0, sem0)
                pltpu.make_async_copy(hs_hbm.at[sb1], rb1, sem1).wait()
                prep_dst(k1, db1, lo)
                pltpu.sync_copy(rb1, agg_sh.at[db1], add=True)
                return carry
            lax.fori_loop(0, (nch - 1) // 2, body, 0)

            pltpu.make_async_copy(hs_hbm.at[sb0], rb0, sem0).wait()
            prep_dst(nch - 1, db0, lo)
            pltpu.sync_copy(rb0, agg_sh.at[db0], add=True)

            plsc.subcore_barrier()

            @pl.when(s < NS - 1)
            def _wmain():
                pltpu.sync_copy(agg_sh.at[pl.ds(s * rows_a, rows_a)],
                                out_hbm.at[pl.ds(lo + s * rows_a, rows_a)])

            @pl.when(s == NS - 1)
            def _wlast():
                pltpu.sync_copy(
                    agg_sh.at[pl.ds((NS - 1) * rows_a, rows_last)],
                    out_hbm.at[pl.ds(lo + (NS - 1) * rows_a, rows_last)])

            plsc.subcore_barrier()

    return prop


def _dinv(cnt):
    deg = cnt + 1.0
    return jnp.where(deg > 0, lax.rsqrt(jnp.maximum(deg, 1e-12)), 0.0)


def _tc_a_body(x_ref, w_ref, cnt_ref, o_ref):
    dinv = _dinv(cnt_ref[...])
    h = jnp.dot(x_ref[...], w_ref[...], preferred_element_type=jnp.float32)
    o_ref[...] = h * dinv


def _tc_b_body(agg_ref, hs_ref, cnt_ref, w_ref, b_ref, o_ref):
    dinv = _dinv(cnt_ref[...])
    z = (agg_ref[...] + hs_ref[...]) * dinv + b_ref[...]
    h = jnp.maximum(z, 0.0)
    o_ref[...] = jnp.dot(h, w_ref[...], preferred_element_type=jnp.float32) * dinv


def _tc_c_body(agg_ref, hs_ref, cnt_ref, b_ref, o_ref):
    dinv = _dinv(cnt_ref[...])
    z = (agg_ref[...] + hs_ref[...]) * dinv + b_ref[...]
    m = jnp.max(z, axis=1, keepdims=True)
    shifted = z - m
    lse = jnp.log(jnp.sum(jnp.exp(shifted), axis=1, keepdims=True))
    o_ref[...] = shifted - lse


def _row_block(n):
    return 1000 if n % 1000 == 0 else 8


def kernel(x, edge_index, W1, b1, W2, b2):
    n, d_in = x.shape
    d_h = W1.shape[1]
    d_out = W2.shape[1]
    e = edge_index.shape[1]
    r = _row_block(n)
    grid = (n // r,)

    src = edge_index[0]
    dst = edge_index[1]
    counts = _make_degree_hist(n, e)(dst)
    cnt = counts.reshape(n, 1)
    b1r = b1.reshape(1, d_h)
    b2r = b2.reshape(1, d_out)

    cnt_spec = pl.BlockSpec((r, 1), lambda i: (i, 0))

    hs1 = pl.pallas_call(
        _tc_a_body,
        grid=grid,
        in_specs=[
            pl.BlockSpec((r, d_in), lambda i: (i, 0)),
            pl.BlockSpec((d_in, d_h), lambda i: (0, 0)),
            cnt_spec,
        ],
        out_specs=pl.BlockSpec((r, d_h), lambda i: (i, 0)),
        out_shape=jax.ShapeDtypeStruct((n, d_h), jnp.float32),
    )(x, W1, cnt)

    agg1 = _make_propagate(n, d_h, e)(hs1, src, dst)

    hs2 = pl.pallas_call(
        _tc_b_body,
        grid=grid,
        in_specs=[
            pl.BlockSpec((r, d_h), lambda i: (i, 0)),
            pl.BlockSpec((r, d_h), lambda i: (i, 0)),
            cnt_spec,
            pl.BlockSpec((d_h, d_out), lambda i: (0, 0)),
            pl.BlockSpec((1, d_h), lambda i: (0, 0)),
        ],
        out_specs=pl.BlockSpec((r, d_out), lambda i: (i, 0)),
        out_shape=jax.ShapeDtypeStruct((n, d_out), jnp.float32),
    )(agg1, hs1, cnt, W2, b1r)

    agg2 = _make_propagate(n, d_out, e)(hs2, src, dst)

    out = pl.pallas_call(
        _tc_c_body,
        grid=grid,
        in_specs=[
            pl.BlockSpec((r, d_out), lambda i: (i, 0)),
            pl.BlockSpec((r, d_out), lambda i: (i, 0)),
            cnt_spec,
            pl.BlockSpec((1, d_out), lambda i: (0, 0)),
        ],
        out_specs=pl.BlockSpec((r, d_out), lambda i: (i, 0)),
        out_shape=jax.ShapeDtypeStruct((n, d_out), jnp.float32),
    )(agg2, hs2, cnt, b2r)

    return out


# speedup vs baseline: 11.2820x; 1.7294x over previous
import functools

import jax
import jax.numpy as jnp
from jax import lax
from jax.experimental import pallas as pl
from jax.experimental.pallas import tpu as pltpu
from jax.experimental.pallas import tpu_sc as plsc

NS = 16
LANES = 16
EBH = 80
EBP = 32


def _sc_mesh(num_cores=1):
    return plsc.VectorSubcoreMesh(core_axis_name="c", subcore_axis_name="s",
                                  num_cores=num_cores)


def _make_degree_hist(n, e):
    ept = e // NS
    nch = ept // EBH

    @functools.partial(
        pl.kernel,
        out_type=jax.ShapeDtypeStruct((n,), jnp.float32),
        mesh=_sc_mesh(),
        scratch_types=[
            pltpu.VMEM((EBH,), jnp.int32),
            pltpu.VMEM((EBH,), jnp.float32),
            pltpu.VMEM((n,), jnp.float32),
            pltpu.VMEM_SHARED((n,), jnp.float32),
        ],
    )
    def hist(dst_hbm, out_hbm, idxb, onesb, zb, counts_sh):
        s = lax.axis_index("s")
        for j in range(EBH // LANES):
            onesb[pl.ds(j * LANES, LANES)] = jnp.ones((LANES,), jnp.float32)

        @pl.when(s == 0)
        def _zero():
            def zf(i, carry):
                zb[pl.ds(i * LANES, LANES)] = jnp.zeros((LANES,), jnp.float32)
                return carry
            lax.fori_loop(0, n // LANES, zf, 0)
            pltpu.sync_copy(zb, counts_sh)

        plsc.subcore_barrier()
        base = s * ept

        def body(k, carry):
            pltpu.sync_copy(dst_hbm.at[pl.ds(base + k * EBH, EBH)], idxb)
            pltpu.sync_copy(onesb, counts_sh.at[idxb], add=True)
            return carry
        lax.fori_loop(0, nch, body, 0)

        plsc.subcore_barrier()

        @pl.when(s == 0)
        def _out():
            pltpu.sync_copy(counts_sh, out_hbm)

    return hist


def _make_propagate(n, d, e):
    ept = e // NS
    nch = ept // EBP
    assert nch % 2 == 1, "pipeline below assumes an odd chunk count"
    nh = n // 2
    rows_a = ((nh + NS - 1) // NS + 7) // 8 * 8
    rows_last = nh - rows_a * (NS - 1)

    @functools.partial(
        pl.kernel,
        out_type=jax.ShapeDtypeStruct((n, d), jnp.float32),
        mesh=_sc_mesh(num_cores=2),
        scratch_types=[
            pltpu.VMEM((ept,), jnp.int32),
            pltpu.VMEM((ept,), jnp.int32),
            pltpu.VMEM((EBP,), jnp.int32),
            pltpu.VMEM((EBP,), jnp.int32),
            pltpu.VMEM((EBP,), jnp.int32),
            pltpu.VMEM((EBP,), jnp.int32),
            pltpu.VMEM((EBP, d), jnp.float32),
            pltpu.VMEM((EBP, d), jnp.float32),
            pltpu.VMEM((rows_a, d), jnp.float32),
            pltpu.VMEM_SHARED((nh + 1, d), jnp.float32),
            pltpu.SemaphoreType.DMA,
            pltpu.SemaphoreType.DMA,
        ],
    )
    def prop(hs_hbm, src_hbm, dst_hbm, out_hbm, srcall, dstall,
             sb0, sb1, db0, db1, rb0, rb1, zb, agg_sh, sem0, sem1):
        c = lax.axis_index("c")
        s = lax.axis_index("s")
        lo = c * nh
        base = s * ept
        pltpu.sync_copy(src_hbm.at[pl.ds(base, ept)], srcall)
        pltpu.sync_copy(dst_hbm.at[pl.ds(base, ept)], dstall)

        def zf(i, carry):
            for j in range(d // LANES):
                zb[i, pl.ds(j * LANES, LANES)] = jnp.zeros((LANES,), jnp.float32)
            return carry
        lax.fori_loop(0, rows_a, zf, 0)

        def prep_src(k, sb):
            for j in range(EBP // LANES):
                sb[pl.ds(j * LANES, LANES)] = srcall[pl.ds(k * EBP + j * LANES,
                                                           LANES)]

        def prep_dst(k, db):
            for j in range(EBP // LANES):
                v = dstall[pl.ds(k * EBP + j * LANES, LANES)]
                local = v - lo
                ok = (local >= 0) & (local < nh)
                db[pl.ds(j * LANES, LANES)] = jnp.where(ok, local, nh)

        @pl.when(s < NS - 1)
        def _zmain():
            pltpu.sync_copy(zb, agg_sh.at[pl.ds(s * rows_a, rows_a)])

        @pl.when(s == NS - 1)
        def _zlast():
            pltpu.sync_copy(zb.at[pl.ds(0, rows_last)],
                            agg_sh.at[pl.ds((NS - 1) * rows_a, rows_last)])

        plsc.subcore_barrier()

        prep_src(0, sb0)
        pltpu.async_copy(hs_hbm.at[sb0], rb0, sem0)

        def body(k2, carry):
            k0 = 2 * k2
            k1 = k0 + 1
            prep_src(k1, sb1)
            pltpu.async_copy(hs_hbm.at[sb1], rb1, sem1)
            pltpu.make_async_copy(hs_hbm.at[sb0], rb0, sem0).wait()
            prep_dst(k0, db0)
            pltpu.sync_copy(rb0, agg_sh.at[db0], add=True)
            prep_src(k0 + 2, sb0)
            pltpu.async_copy(hs_hbm.at[sb0], rb0, sem0)
            pltpu.make_async_copy(hs_hbm.at[sb1], rb1, sem1).wait()
            prep_dst(k1, db1)
            pltpu.sync_copy(rb1, agg_sh.at[db1], add=True)
            return carry
        lax.fori_loop(0, (nch - 1) // 2, body, 0)

        pltpu.make_async_copy(hs_hbm.at[sb0], rb0, sem0).wait()
        prep_dst(nch - 1, db0)
        pltpu.sync_copy(rb0, agg_sh.at[db0], add=True)

        plsc.subcore_barrier()

        @pl.when(s < NS - 1)
        def _wmain():
            pltpu.sync_copy(agg_sh.at[pl.ds(s * rows_a, rows_a)],
                            out_hbm.at[pl.ds(lo + s * rows_a, rows_a)])

        @pl.when(s == NS - 1)
        def _wlast():
            pltpu.sync_copy(
                agg_sh.at[pl.ds((NS - 1) * rows_a, rows_last)],
                out_hbm.at[pl.ds(lo + (NS - 1) * rows_a, rows_last)])

    return prop


def _dinv(cnt):
    deg = cnt + 1.0
    return jnp.where(deg > 0, lax.rsqrt(jnp.maximum(deg, 1e-12)), 0.0)


def _tc_a_body(x_ref, w_ref, cnt_ref, o_ref):
    dinv = _dinv(cnt_ref[...])
    h = jnp.dot(x_ref[...], w_ref[...], preferred_element_type=jnp.float32)
    o_ref[...] = h * dinv


def _tc_b_body(agg_ref, hs_ref, cnt_ref, w_ref, b_ref, o_ref):
    dinv = _dinv(cnt_ref[...])
    z = (agg_ref[...] + hs_ref[...]) * dinv + b_ref[...]
    h = jnp.maximum(z, 0.0)
    o_ref[...] = jnp.dot(h, w_ref[...], preferred_element_type=jnp.float32) * dinv


def _tc_c_body(agg_ref, hs_ref, cnt_ref, b_ref, o_ref):
    dinv = _dinv(cnt_ref[...])
    z = (agg_ref[...] + hs_ref[...]) * dinv + b_ref[...]
    m = jnp.max(z, axis=1, keepdims=True)
    shifted = z - m
    lse = jnp.log(jnp.sum(jnp.exp(shifted), axis=1, keepdims=True))
    o_ref[...] = shifted - lse


def _row_block(n):
    return 1000 if n % 1000 == 0 else 8


def kernel(x, edge_index, W1, b1, W2, b2):
    n, d_in = x.shape
    d_h = W1.shape[1]
    d_out = W2.shape[1]
    e = edge_index.shape[1]
    r = _row_block(n)
    grid = (n // r,)

    src = edge_index[0]
    dst = edge_index[1]
    counts = _make_degree_hist(n, e)(dst)
    cnt = counts.reshape(n, 1)
    b1r = b1.reshape(1, d_h)
    b2r = b2.reshape(1, d_out)

    cnt_spec = pl.BlockSpec((r, 1), lambda i: (i, 0))

    hs1 = pl.pallas_call(
        _tc_a_body,
        grid=grid,
        in_specs=[
            pl.BlockSpec((r, d_in), lambda i: (i, 0)),
            pl.BlockSpec((d_in, d_h), lambda i: (0, 0)),
            cnt_spec,
        ],
        out_specs=pl.BlockSpec((r, d_h), lambda i: (i, 0)),
        out_shape=jax.ShapeDtypeStruct((n, d_h), jnp.float32),
    )(x, W1, cnt)

    agg1 = _make_propagate(n, d_h, e)(hs1, src, dst)

    hs2 = pl.pallas_call(
        _tc_b_body,
        grid=grid,
        in_specs=[
            pl.BlockSpec((r, d_h), lambda i: (i, 0)),
            pl.BlockSpec((r, d_h), lambda i: (i, 0)),
            cnt_spec,
            pl.BlockSpec((d_h, d_out), lambda i: (0, 0)),
            pl.BlockSpec((1, d_h), lambda i: (0, 0)),
        ],
        out_specs=pl.BlockSpec((r, d_out), lambda i: (i, 0)),
        out_shape=jax.ShapeDtypeStruct((n, d_out), jnp.float32),
    )(agg1, hs1, cnt, W2, b1r)

    agg2 = _make_propagate(n, d_out, e)(hs2, src, dst)

    out = pl.pallas_call(
        _tc_c_body,
        grid=grid,
        in_specs=[
            pl.BlockSpec((r, d_out), lambda i: (i, 0)),
            pl.BlockSpec((r, d_out), lambda i: (i, 0)),
            cnt_spec,
            pl.BlockSpec((1, d_out), lambda i: (0, 0)),
        ],
        out_specs=pl.BlockSpec((r, d_out), lambda i: (i, 0)),
        out_shape=jax.ShapeDtypeStruct((n, d_out), jnp.float32),
    )(agg2, hs2, cnt, b2r)

    return out


# speedup vs baseline: 12.7955x; 1.1342x over previous
import functools

import jax
import jax.numpy as jnp
from jax import lax
from jax.experimental import pallas as pl
from jax.experimental.pallas import tpu as pltpu
from jax.experimental.pallas import tpu_sc as plsc

NS = 16
LANES = 16
EBH = 80
EBP = 32


def _sc_mesh(num_cores=1):
    return plsc.VectorSubcoreMesh(core_axis_name="c", subcore_axis_name="s",
                                  num_cores=num_cores)


def _make_degree_hist(n, e):
    epc = e // 2
    ept = epc // NS
    nch = ept // EBH

    @functools.partial(
        pl.kernel,
        out_type=(jax.ShapeDtypeStruct((n,), jnp.float32),
                  jax.ShapeDtypeStruct((n,), jnp.float32)),
        mesh=_sc_mesh(num_cores=2),
        scratch_types=[
            pltpu.VMEM((ept,), jnp.int32),
            pltpu.VMEM((EBH,), jnp.int32),
            pltpu.VMEM((EBH,), jnp.float32),
            pltpu.VMEM((n,), jnp.float32),
            pltpu.VMEM_SHARED((n,), jnp.float32),
        ],
    )
    def hist(dst_hbm, out0_hbm, out1_hbm, dstall, idxb, onesb, zb, counts_sh):
        c = lax.axis_index("c")
        s = lax.axis_index("s")
        base = c * epc + s * ept
        pltpu.sync_copy(dst_hbm.at[pl.ds(base, ept)], dstall)
        for j in range(EBH // LANES):
            onesb[pl.ds(j * LANES, LANES)] = jnp.ones((LANES,), jnp.float32)

        @pl.when(s == 0)
        def _zero():
            def zf(i, carry):
                zb[pl.ds(i * LANES, LANES)] = jnp.zeros((LANES,), jnp.float32)
                return carry
            lax.fori_loop(0, n // LANES, zf, 0)
            pltpu.sync_copy(zb, counts_sh)

        plsc.subcore_barrier()

        def body(k, carry):
            for j in range(EBH // LANES):
                idxb[pl.ds(j * LANES, LANES)] = dstall[
                    pl.ds(k * EBH + j * LANES, LANES)]
            pltpu.sync_copy(onesb, counts_sh.at[idxb], add=True)
            return carry
        lax.fori_loop(0, nch, body, 0)

        plsc.subcore_barrier()

        @pl.when((s == 0) & (c == 0))
        def _out0():
            pltpu.sync_copy(counts_sh, out0_hbm)

        @pl.when((s == 0) & (c == 1))
        def _out1():
            pltpu.sync_copy(counts_sh, out1_hbm)

    return hist


def _make_propagate(n, d, e):
    ept = e // NS
    nch = ept // EBP
    assert nch % 2 == 1, "pipeline below assumes an odd chunk count"
    nh = n // 2
    rows_a = ((nh + NS - 1) // NS + 7) // 8 * 8
    rows_last = nh - rows_a * (NS - 1)

    @functools.partial(
        pl.kernel,
        out_type=jax.ShapeDtypeStruct((n, d), jnp.float32),
        mesh=_sc_mesh(num_cores=2),
        scratch_types=[
            pltpu.VMEM((ept,), jnp.int32),
            pltpu.VMEM((ept,), jnp.int32),
            pltpu.VMEM((EBP,), jnp.int32),
            pltpu.VMEM((EBP,), jnp.int32),
            pltpu.VMEM((EBP,), jnp.int32),
            pltpu.VMEM((EBP,), jnp.int32),
            pltpu.VMEM((EBP, d), jnp.float32),
            pltpu.VMEM((EBP, d), jnp.float32),
            pltpu.VMEM((rows_a, d), jnp.float32),
            pltpu.VMEM_SHARED((nh + 1, d), jnp.float32),
            pltpu.SemaphoreType.DMA,
            pltpu.SemaphoreType.DMA,
        ],
    )
    def prop(hs_hbm, src_hbm, dst_hbm, out_hbm, srcall, dstall,
             sb0, sb1, db0, db1, rb0, rb1, zb, agg_sh, sem0, sem1):
        c = lax.axis_index("c")
        s = lax.axis_index("s")
        lo = c * nh
        base = s * ept
        pltpu.sync_copy(src_hbm.at[pl.ds(base, ept)], srcall)
        pltpu.sync_copy(dst_hbm.at[pl.ds(base, ept)], dstall)

        def zf(i, carry):
            for j in range(d // LANES):
                zb[i, pl.ds(j * LANES, LANES)] = jnp.zeros((LANES,), jnp.float32)
            return carry
        lax.fori_loop(0, rows_a, zf, 0)

        def prep_src(k, sb):
            for j in range(EBP // LANES):
                sb[pl.ds(j * LANES, LANES)] = srcall[pl.ds(k * EBP + j * LANES,
                                                           LANES)]

        def prep_dst(k, db):
            for j in range(EBP // LANES):
                v = dstall[pl.ds(k * EBP + j * LANES, LANES)]
                local = v - lo
                ok = (local >= 0) & (local < nh)
                db[pl.ds(j * LANES, LANES)] = jnp.where(ok, local, nh)

        @pl.when(s < NS - 1)
        def _zmain():
            pltpu.sync_copy(zb, agg_sh.at[pl.ds(s * rows_a, rows_a)])

        @pl.when(s == NS - 1)
        def _zlast():
            pltpu.sync_copy(zb.at[pl.ds(0, rows_last)],
                            agg_sh.at[pl.ds((NS - 1) * rows_a, rows_last)])

        plsc.subcore_barrier()

        prep_src(0, sb0)
        pltpu.async_copy(hs_hbm.at[sb0], rb0, sem0)

        def body(k2, carry):
            k0 = 2 * k2
            k1 = k0 + 1
            prep_src(k1, sb1)
            pltpu.async_copy(hs_hbm.at[sb1], rb1, sem1)
            pltpu.make_async_copy(hs_hbm.at[sb0], rb0, sem0).wait()
            prep_dst(k0, db0)
            pltpu.sync_copy(rb0, agg_sh.at[db0], add=True)
            prep_src(k0 + 2, sb0)
            pltpu.async_copy(hs_hbm.at[sb0], rb0, sem0)
            pltpu.make_async_copy(hs_hbm.at[sb1], rb1, sem1).wait()
            prep_dst(k1, db1)
            pltpu.sync_copy(rb1, agg_sh.at[db1], add=True)
            return carry
        lax.fori_loop(0, (nch - 1) // 2, body, 0)

        pltpu.make_async_copy(hs_hbm.at[sb0], rb0, sem0).wait()
        prep_dst(nch - 1, db0)
        pltpu.sync_copy(rb0, agg_sh.at[db0], add=True)

        plsc.subcore_barrier()

        @pl.when(s < NS - 1)
        def _wmain():
            pltpu.sync_copy(agg_sh.at[pl.ds(s * rows_a, rows_a)],
                            out_hbm.at[pl.ds(lo + s * rows_a, rows_a)])

        @pl.when(s == NS - 1)
        def _wlast():
            pltpu.sync_copy(
                agg_sh.at[pl.ds((NS - 1) * rows_a, rows_last)],
                out_hbm.at[pl.ds(lo + (NS - 1) * rows_a, rows_last)])

    return prop


def _dinv(c0, c1):
    deg = c0 + c1 + 1.0
    return jnp.where(deg > 0, lax.rsqrt(jnp.maximum(deg, 1e-12)), 0.0)


def _tc_a_body(x_ref, w_ref, c0_ref, c1_ref, o_ref):
    dinv = _dinv(c0_ref[...], c1_ref[...])
    h = jnp.dot(x_ref[...], w_ref[...], preferred_element_type=jnp.float32)
    o_ref[...] = h * dinv


def _tc_b_body(agg_ref, hs_ref, c0_ref, c1_ref, w_ref, b_ref, o_ref):
    dinv = _dinv(c0_ref[...], c1_ref[...])
    z = (agg_ref[...] + hs_ref[...]) * dinv + b_ref[...]
    h = jnp.maximum(z, 0.0)
    o_ref[...] = jnp.dot(h, w_ref[...], preferred_element_type=jnp.float32) * dinv


def _tc_c_body(agg_ref, hs_ref, c0_ref, c1_ref, b_ref, o_ref):
    dinv = _dinv(c0_ref[...], c1_ref[...])
    z = (agg_ref[...] + hs_ref[...]) * dinv + b_ref[...]
    m = jnp.max(z, axis=1, keepdims=True)
    shifted = z - m
    lse = jnp.log(jnp.sum(jnp.exp(shifted), axis=1, keepdims=True))
    o_ref[...] = shifted - lse


def _row_block(n):
    return 1000 if n % 1000 == 0 else 8


def kernel(x, edge_index, W1, b1, W2, b2):
    n, d_in = x.shape
    d_h = W1.shape[1]
    d_out = W2.shape[1]
    e = edge_index.shape[1]
    r = _row_block(n)
    grid = (n // r,)

    src = edge_index[0]
    dst = edge_index[1]
    counts0, counts1 = _make_degree_hist(n, e)(dst)
    c0 = counts0.reshape(n, 1)
    c1 = counts1.reshape(n, 1)
    b1r = b1.reshape(1, d_h)
    b2r = b2.reshape(1, d_out)

    cnt_spec = pl.BlockSpec((r, 1), lambda i: (i, 0))

    hs1 = pl.pallas_call(
        _tc_a_body,
        grid=grid,
        in_specs=[
            pl.BlockSpec((r, d_in), lambda i: (i, 0)),
            pl.BlockSpec((d_in, d_h), lambda i: (0, 0)),
            cnt_spec, cnt_spec,
        ],
        out_specs=pl.BlockSpec((r, d_h), lambda i: (i, 0)),
        out_shape=jax.ShapeDtypeStruct((n, d_h), jnp.float32),
    )(x, W1, c0, c1)

    agg1 = _make_propagate(n, d_h, e)(hs1, src, dst)

    hs2 = pl.pallas_call(
        _tc_b_body,
        grid=grid,
        in_specs=[
            pl.BlockSpec((r, d_h), lambda i: (i, 0)),
            pl.BlockSpec((r, d_h), lambda i: (i, 0)),
            cnt_spec, cnt_spec,
            pl.BlockSpec((d_h, d_out), lambda i: (0, 0)),
            pl.BlockSpec((1, d_h), lambda i: (0, 0)),
        ],
        out_specs=pl.BlockSpec((r, d_out), lambda i: (i, 0)),
        out_shape=jax.ShapeDtypeStruct((n, d_out), jnp.float32),
    )(agg1, hs1, c0, c1, W2, b1r)

    agg2 = _make_propagate(n, d_out, e)(hs2, src, dst)

    out = pl.pallas_call(
        _tc_c_body,
        grid=grid,
        in_specs=[
            pl.BlockSpec((r, d_out), lambda i: (i, 0)),
            pl.BlockSpec((r, d_out), lambda i: (i, 0)),
            cnt_spec, cnt_spec,
            pl.BlockSpec((1, d_out), lambda i: (0, 0)),
        ],
        out_specs=pl.BlockSpec((r, d_out), lambda i: (i, 0)),
        out_shape=jax.ShapeDtypeStruct((n, d_out), jnp.float32),
    )(agg2, hs2, c0, c1, b2r)

    return out
